# Initial kernel scaffold; baseline (speedup 1.0000x reference)
import jax
import jax.numpy as jnp
from jax.experimental import pallas as pl


def kernel(query, keys, values, Wq, bq, Wo, bo, gamma):
    raise NotImplementedError("write your pallas kernel here")


# speedup vs baseline: 1.3877x; 1.3877x over previous
import numpy as np

import jax
import jax.numpy as jnp
from jax import lax
from jax.experimental import pallas as pl
from jax.experimental.pallas import tpu as pltpu
from jax.experimental.pallas import tpu_sc as plsc

QD = 64
D2 = 256
N = 100000
K = 64
CHUNK = 16384
NSTEPS = 7
NPAD = CHUNK * NSTEPS
NBLK = NPAD // 128
CAP_B = 128
CAPC = 2048
NCORE = 2
NSUB = 16
NW = NCORE * NSUB
QPT = QD // NW
MINI32 = np.int32(-2147483648)
MAXI32 = np.int32(2147483647)


def _mono(ib):
    return jnp.where(ib < 0, ib ^ MAXI32, ib)


def _tc1_body(x_ref, w_ref, b_ref, keys_ref, coh_ref, bm_ref, aux_ref, q_scr):
    i = pl.program_id(0)

    @pl.when(i == 0)
    def _q():
        q_scr[...] = (
            jnp.dot(x_ref[...], w_ref[...], preferred_element_type=jnp.float32)
            + b_ref[...]
        )

    qil = q_scr[...]
    qmag = jnp.sqrt(jnp.sum(qil * qil, axis=1, keepdims=True) + 1e-8)
    kblk = keys_ref[...]
    coh = lax.dot_general(
        qil, kblk, (((1,), (1,)), ((), ())), preferred_element_type=jnp.float32
    )
    ones8 = jnp.ones((8, D2), jnp.float32)
    km2 = lax.dot_general(
        ones8, kblk * kblk, (((1,), (1,)), ((), ())),
        precision=lax.Precision.HIGHEST,
        preferred_element_type=jnp.float32,
    )
    kmag = jnp.sqrt(km2[0:1, :] + 1e-8)
    coh = coh / (qmag * kmag + 1e-8)
    col = i * CHUNK + lax.broadcasted_iota(jnp.int32, (QD, CHUNK), 1)
    coh = jnp.where(col < N, coh, -jnp.inf)
    coh_ref[...] = coh
    bm_ref[:, pl.ds(i * (CHUNK // 128), CHUNK // 128)] = jnp.max(
        coh.reshape(QD, CHUNK // 128, 128), axis=2
    )

    @pl.when(i == NSTEPS - 1)
    def _fin():
        key = _mono(lax.bitcast_convert_type(bm_ref[...], jnp.int32))

        def bit(jb, t):
            cu = t | (jnp.int32(1) << (31 - jb))
            cs = cu ^ MINI32
            cnt = jnp.sum((key >= cs).astype(jnp.int32), axis=1, keepdims=True)
            return jnp.where(cnt >= K, cu, t)

        tu = lax.fori_loop(0, 32, bit, jnp.zeros((QD, 1), jnp.int32))
        thr = lax.bitcast_convert_type(_mono(tu ^ MINI32), jnp.float32)
        aux_ref[...] = jnp.broadcast_to(thr, (QD, 128))


def _tc2_body(r_ref, w_ref, b_ref, g_ref, o_ref):
    r = r_ref[:, :D2] / r_ref[:, D2:D2 + 1]
    ssum = jnp.sum(r * r, axis=1, keepdims=True)
    rms = jnp.sqrt(ssum / 128.0 + 1e-8 + 1e-5)
    xn = r / rms * g_ref[...]
    o_ref[...] = (
        jnp.dot(xn, w_ref[...], preferred_element_type=jnp.float32) + b_ref[...]
    )


def _sc_body(coh_hbm, bm_hbm, aux_hbm, vals_hbm, out_hbm,
             thrv, bmv, blk_idx, g0, g1, g2, g3, g4, g5, g6, g7,
             cand_val, cand_idx, key_buf,
             sel_val, sel_idx, r0, r1, r2, r3, acc_buf, sem):
    wid = lax.axis_index("s") * NCORE + lax.axis_index("c")
    lane = lax.iota(jnp.int32, 16)
    gbufs = (g0, g1, g2, g3, g4, g5, g6, g7)
    rbufs = (r0, r1, r2, r3)

    for qi in range(QPT):
        q = wid * QPT + qi

        pltpu.sync_copy(aux_hbm.at[q], thrv)
        thr_s = jnp.max(thrv[pl.ds(0, 16)])
        thr_vec = jnp.broadcast_to(thr_s, (16,))

        pltpu.sync_copy(bm_hbm.at[q], bmv)
        safe_blk = jnp.broadcast_to(q * NBLK, (16,)).astype(jnp.int32)
        for t in range(CAP_B // 16):
            blk_idx[pl.ds(t * 16, 16)] = safe_blk
        cnt_b = jnp.int32(0)
        for i in range(NBLK // 16):
            v = bmv[pl.ds(i * 16, 16)]
            m = v >= thr_vec
            gidx = q * NBLK + i * 16 + lane
            plsc.store_compressed(blk_idx.at[pl.ds(cnt_b, 16)], gidx, mask=m)
            cnt_b = jnp.minimum(
                cnt_b + jnp.sum(m.astype(jnp.int32)), jnp.int32(CAP_B - 16)
            )
        blk_idx[pl.ds(cnt_b, 16)] = safe_blk

        ivs = [blk_idx[pl.ds(g * 16, 16)] for g in range(8)]
        descs = [
            pltpu.async_copy(coh_hbm.at[ivs[g]], gbufs[g], sem)
            for g in range(8)
        ]
        for d in descs:
            d.wait()

        coff = jnp.int32(0)
        for g in range(8):
            hi = jnp.clip(cnt_b - g * 16, 0, 16)
            bv = ivs[g]

            def blk_body(j, coff, g=g, bv=bv):
                base = jnp.max(jnp.where(lane == j, bv, MINI32))
                nbase = (base - q * NBLK) * 128
                for k8 in range(8):
                    v = gbufs[g][j, pl.ds(k8 * 16, 16)]
                    m = v >= thr_vec
                    plsc.store_compressed(
                        cand_val.at[pl.ds(coff, 16)], v, mask=m)
                    iv = nbase + k8 * 16 + lane
                    plsc.store_compressed(
                        cand_idx.at[pl.ds(coff, 16)], iv, mask=m)
                    coff = jnp.minimum(
                        coff + jnp.sum(m.astype(jnp.int32)),
                        jnp.int32(CAPC - 16))
                return coff

            coff = lax.fori_loop(0, hi, blk_body, coff)
        cand_val[pl.ds(coff, 16)] = jnp.broadcast_to(-jnp.inf, (16,))
        nvec = lax.shift_right_logical(coff, 4) + 1

        def conv(v, _):
            x = cand_val[pl.ds(v * 16, 16)]
            key_buf[pl.ds(v * 16, 16)] = _mono(
                lax.bitcast_convert_type(x, jnp.int32)
            )
            return 0

        lax.fori_loop(0, nvec, conv, 0)

        def bit(jb, t):
            cu = t | (jnp.int32(1) << (31 - jb))
            cs = jnp.broadcast_to(cu ^ MINI32, (16,))

            def cbody(v, a):
                return a + jnp.sum(
                    (key_buf[pl.ds(v * 16, 16)] >= cs).astype(jnp.int32)
                )

            cnt = lax.fori_loop(0, nvec, cbody, jnp.int32(0))
            return jnp.where(cnt >= K, cu, t)

        tu = lax.fori_loop(0, 32, bit, jnp.int32(0))
        tkey = tu ^ MINI32
        tkv = jnp.broadcast_to(tkey, (16,))

        def gbody(v, a):
            return a + jnp.sum((key_buf[pl.ds(v * 16, 16)] > tkv).astype(jnp.int32))

        c1 = lax.fori_loop(0, nvec, gbody, jnp.int32(0))
        quota = K - c1

        def sel(v, car):
            soff, tc = car
            kv = key_buf[pl.ds(v * 16, 16)]
            gt = kv > tkv
            eq = kv == tkv
            cs = plsc.cumsum(eq.astype(jnp.int32))
            m = gt | (eq & ((tc + cs) <= quota))
            plsc.store_compressed(
                sel_val.at[pl.ds(soff, 16)], cand_val[pl.ds(v * 16, 16)], mask=m
            )
            plsc.store_compressed(
                sel_idx.at[pl.ds(soff, 16)], cand_idx[pl.ds(v * 16, 16)], mask=m
            )
            return (
                soff + jnp.sum(m.astype(jnp.int32)),
                tc + jnp.sum(eq.astype(jnp.int32)),
            )

        for t in range(4):
            sel_idx[pl.ds(t * 16, 16)] = jnp.broadcast_to(jnp.int32(0), (16,))
        lax.fori_loop(0, nvec, sel, (jnp.int32(0), jnp.int32(0)))

        sivs = [sel_idx[pl.ds(t * 16, 16)] for t in range(4)]
        rdescs = [
            pltpu.async_copy(vals_hbm.at[sivs[t]], rbufs[t], sem)
            for t in range(4)
        ]

        svs = [sel_val[pl.ds(t * 16, 16)] for t in range(4)]
        m1 = jnp.maximum(
            jnp.maximum(jnp.max(svs[0]), jnp.max(svs[1])),
            jnp.maximum(jnp.max(svs[2]), jnp.max(svs[3])),
        )
        m1v = jnp.broadcast_to(m1, (16,))
        wvs = [jnp.exp(sv - m1v) for sv in svs]
        z = (jnp.sum(wvs[0]) + jnp.sum(wvs[1])
             + jnp.sum(wvs[2]) + jnp.sum(wvs[3]))

        for d in rdescs:
            d.wait()

        accs = tuple(jnp.zeros((16,), jnp.float32) for _ in range(16))
        for t in range(4):
            wv = wvs[t]

            def kbody(k, accs, t=t, wv=wv):
                wk = jnp.max(jnp.where(lane == k, wv, -jnp.inf))
                wkv = jnp.broadcast_to(wk, (16,))
                return tuple(
                    accs[j] + rbufs[t][k, pl.ds(j * 16, 16)] * wkv
                    for j in range(16)
                )

            accs = lax.fori_loop(0, 16, kbody, accs)
        for j in range(16):
            acc_buf[pl.ds(j * 16, 16)] = accs[j]
        acc_buf[pl.ds(D2, 16)] = jnp.broadcast_to(z, (16,))
        pltpu.sync_copy(acc_buf, out_hbm.at[q])


def _sc_call(coh_rows, bm, aux, vals2):
    mesh = plsc.VectorSubcoreMesh(
        core_axis_name="c", subcore_axis_name="s",
        num_cores=NCORE, num_subcores=NSUB,
    )
    return pl.kernel(
        _sc_body,
        out_type=jax.ShapeDtypeStruct((QD, D2 + 16), jnp.float32),
        mesh=mesh,
        compiler_params=pltpu.CompilerParams(needs_layout_passes=False),
        scratch_types=[
            pltpu.VMEM((128,), jnp.float32),
            pltpu.VMEM((NBLK,), jnp.float32),
            pltpu.VMEM((CAP_B,), jnp.int32),
        ] + [pltpu.VMEM((16, 128), jnp.float32) for _ in range(8)]
        + [
            pltpu.VMEM((CAPC + 16,), jnp.float32),
            pltpu.VMEM((CAPC + 16,), jnp.int32),
            pltpu.VMEM((CAPC + 16,), jnp.int32),
            pltpu.VMEM((K + 16,), jnp.float32),
            pltpu.VMEM((K + 16,), jnp.int32),
        ] + [pltpu.VMEM((16, D2), jnp.float32) for _ in range(4)]
        + [
            pltpu.VMEM((D2 + 16,), jnp.float32),
            pltpu.SemaphoreType.DMA,
        ],
    )(coh_rows, bm, aux, vals2)


def _interleave_w(wr, wi):
    a = jnp.stack([jnp.stack([wr, wi], -1), jnp.stack([-wi, wr], -1)], 1)
    return a.reshape(2 * wr.shape[0], 2 * wr.shape[1])


@jax.jit
def kernel(query, keys, values, Wq, bq, Wo, bo, gamma):
    x64 = query.reshape(QD, D2)
    kil = keys.reshape(N, D2)
    vals2 = values.reshape(N, D2)
    w2q = _interleave_w(Wq[..., 0], Wq[..., 1])
    b2q = bq.reshape(1, D2)
    w2o = _interleave_w(Wo[..., 0], Wo[..., 1])
    b2o = bo.reshape(1, D2)
    gil = jnp.stack([gamma, gamma], -1).reshape(1, D2)

    coh, bm, aux = pl.pallas_call(
        _tc1_body,
        grid=(NSTEPS,),
        in_specs=[
            pl.BlockSpec((QD, D2), lambda i: (0, 0)),
            pl.BlockSpec((D2, D2), lambda i: (0, 0)),
            pl.BlockSpec((1, D2), lambda i: (0, 0)),
            pl.BlockSpec((CHUNK, D2), lambda i: (i, 0)),
        ],
        out_specs=[
            pl.BlockSpec((QD, CHUNK), lambda i: (0, i)),
            pl.BlockSpec((QD, NBLK), lambda i: (0, 0)),
            pl.BlockSpec((QD, 128), lambda i: (0, 0)),
        ],
        out_shape=[
            jax.ShapeDtypeStruct((QD, NPAD), jnp.float32),
            jax.ShapeDtypeStruct((QD, NBLK), jnp.float32),
            jax.ShapeDtypeStruct((QD, 128), jnp.float32),
        ],
        scratch_shapes=[pltpu.VMEM((QD, D2), jnp.float32)],
    )(x64, w2q, b2q, kil)

    coh_rows = coh.reshape(QD * NBLK, 128)
    retr = _sc_call(coh_rows, bm, aux, vals2)

    out64 = pl.pallas_call(
        _tc2_body,
        out_shape=jax.ShapeDtypeStruct((QD, D2), jnp.float32),
    )(retr, w2o, b2o, gil)

    return out64.reshape(query.shape)
